# baseline (speedup 1.0000x reference)
import functools

import jax
import jax.numpy as jnp
from jax import lax
from jax.experimental import pallas as pl


def _fused_lora_body(slot_ref, scale_ref, x_ref, a_ref, b_ref, o_ref, *, rank):
    xb = x_ref[...].astype(jnp.bfloat16)
    h = jnp.dot(xb, a_ref[...], preferred_element_type=jnp.float32)
    slots = slot_ref[...]
    er = h.shape[1]
    col_slot = lax.broadcasted_iota(jnp.int32, (h.shape[0], er), 1) // rank
    h = jnp.where(col_slot == slots, h * scale_ref[...], 0.0)
    o_ref[...] = jnp.dot(h.astype(jnp.bfloat16), b_ref[...],
                         preferred_element_type=jnp.float32)


def kernel(x, token_to_slot, lora_a, lora_b, lora_scaling):
    T, D = x.shape
    E, _, R = lora_a.shape
    D_out = lora_b.shape[-1]
    ER = E * R

    a_cat = jnp.transpose(lora_a, (1, 0, 2)).reshape(D, ER).astype(jnp.bfloat16)
    b_cat = lora_b.reshape(ER, D_out).astype(jnp.bfloat16)
    scale_vec = jnp.repeat(lora_scaling, R).reshape(1, ER)
    slots2 = token_to_slot.reshape(T, 1).astype(jnp.int32)

    BT = 1536
    grid = (pl.cdiv(T, BT),)

    return pl.pallas_call(
        functools.partial(_fused_lora_body, rank=R),
        grid=grid,
        in_specs=[
            pl.BlockSpec((BT, 1), lambda i: (i, 0)),
            pl.BlockSpec((1, ER), lambda i: (0, 0)),
            pl.BlockSpec((BT, D), lambda i: (i, 0)),
            pl.BlockSpec((D, ER), lambda i: (0, 0)),
            pl.BlockSpec((ER, D_out), lambda i: (0, 0)),
        ],
        out_specs=pl.BlockSpec((BT, D_out), lambda i: (i, 0)),
        out_shape=jax.ShapeDtypeStruct((T, D_out), x.dtype),
    )(slots2, scale_vec, x, a_cat, b_cat)


# speedup vs baseline: 1.0091x; 1.0091x over previous
import functools

import jax
import jax.numpy as jnp
from jax import lax
from jax.experimental import pallas as pl
from jax.experimental.pallas import tpu as pltpu


def _fused_lora_body(slot_ref, scale_ref, x_ref, a_ref, b_ref, o_ref, *, rank):
    h = jnp.dot(x_ref[...], a_ref[...], preferred_element_type=jnp.float32)
    slots = slot_ref[...]
    er = h.shape[1]
    col_slot = lax.broadcasted_iota(jnp.int32, (h.shape[0], er), 1) // rank
    h = jnp.where(col_slot == slots, h * scale_ref[...], 0.0)
    o_ref[...] = jnp.dot(h, b_ref[...], preferred_element_type=jnp.float32)


def kernel(x, token_to_slot, lora_a, lora_b, lora_scaling):
    T, D = x.shape
    E, _, R = lora_a.shape
    D_out = lora_b.shape[-1]
    ER = E * R

    a_cat = jnp.transpose(lora_a, (1, 0, 2)).reshape(D, ER)
    b_cat = lora_b.reshape(ER, D_out)
    scale_vec = jnp.repeat(lora_scaling, R).reshape(1, ER)
    slots2 = token_to_slot.reshape(T, 1).astype(jnp.int32)

    BT = 1536
    grid = (pl.cdiv(T, BT),)

    return pl.pallas_call(
        functools.partial(_fused_lora_body, rank=R),
        grid=grid,
        in_specs=[
            pl.BlockSpec((BT, 1), lambda i: (i, 0)),
            pl.BlockSpec((1, ER), lambda i: (0, 0)),
            pl.BlockSpec((BT, D), lambda i: (i, 0)),
            pl.BlockSpec((D, ER), lambda i: (0, 0)),
            pl.BlockSpec((ER, D_out), lambda i: (0, 0)),
        ],
        out_specs=pl.BlockSpec((BT, D_out), lambda i: (i, 0)),
        out_shape=jax.ShapeDtypeStruct((T, D_out), x.dtype),
        compiler_params=pltpu.CompilerParams(
            dimension_semantics=("parallel",),
        ),
    )(slots2, scale_vec, x, a_cat, b_cat)


# speedup vs baseline: 1.0168x; 1.0077x over previous
import functools

import jax
import jax.numpy as jnp
from jax import lax
from jax.experimental import pallas as pl
from jax.experimental.pallas import tpu as pltpu


def _fused_lora_body(slot_ref, scale_ref, x_ref, a_ref, b_ref, o_ref, *, rank):
    h = jnp.dot(x_ref[...], a_ref[...], preferred_element_type=jnp.float32)
    slots = slot_ref[...]
    er = h.shape[1]
    col_slot = lax.broadcasted_iota(jnp.int32, (h.shape[0], er), 1) // rank
    h = jnp.where(col_slot == slots, h * scale_ref[...], 0.0)
    o_ref[...] = jnp.dot(h, b_ref[...], preferred_element_type=jnp.float32)


def kernel(x, token_to_slot, lora_a, lora_b, lora_scaling):
    T, D = x.shape
    E, _, R = lora_a.shape
    D_out = lora_b.shape[-1]
    ER = E * R

    a_cat = jnp.transpose(lora_a, (1, 0, 2)).reshape(D, ER)
    b_cat = lora_b.reshape(ER, D_out)
    scale_vec = jnp.repeat(lora_scaling, R).reshape(1, ER)
    slots2 = token_to_slot.reshape(T, 1).astype(jnp.int32)

    BT = 1664
    grid = (pl.cdiv(T, BT),)

    return pl.pallas_call(
        functools.partial(_fused_lora_body, rank=R),
        grid=grid,
        in_specs=[
            pl.BlockSpec((BT, 1), lambda i: (i, 0)),
            pl.BlockSpec((1, ER), lambda i: (0, 0)),
            pl.BlockSpec((BT, D), lambda i: (i, 0)),
            pl.BlockSpec((D, ER), lambda i: (0, 0)),
            pl.BlockSpec((ER, D_out), lambda i: (0, 0)),
        ],
        out_specs=pl.BlockSpec((BT, D_out), lambda i: (i, 0)),
        out_shape=jax.ShapeDtypeStruct((T, D_out), x.dtype),
        compiler_params=pltpu.CompilerParams(
            dimension_semantics=("parallel",),
        ),
    )(slots2, scale_vec, x, a_cat, b_cat)
